# Initial kernel scaffold; baseline (speedup 1.0000x reference)
import jax
import jax.numpy as jnp
from jax.experimental import pallas as pl


def kernel(x, pos, n_sampling, W1, b1, g1, beta1, W2, b2, g2, beta2):
    raise NotImplementedError("write your pallas kernel here")


# speedup vs baseline: 9.6462x; 9.6462x over previous
import functools

import jax
import jax.numpy as jnp
from jax import lax
from jax.experimental import pallas as pl
from jax.experimental.pallas import tpu as pltpu
from jax.experimental.pallas import tpu_sc as plsc

KNB = 16
NPTS = 2048
BZ = 4
EPSV = 1e-5
F32 = jnp.float32


def _knn_body(q_ref, aT_ref, o_ref):
    b = pl.program_id(0)
    pq = q_ref[0]
    pa = aT_ref[0]
    dx = pq[:, 0:1] - pa[0:1, :]
    dy = pq[:, 1:2] - pa[1:2, :]
    dz = pq[:, 2:3] - pa[2:3, :]
    d = (dx * dx + dy * dy) + dz * dz
    R = d.shape[0]
    iota = lax.broadcasted_iota(jnp.int32, (R, NPTS), 1)
    cols = []
    for _ in range(KNB):
        m = jnp.min(d, axis=1, keepdims=True)
        im = jnp.min(jnp.where(d == m, iota, NPTS), axis=1, keepdims=True)
        cols.append(im)
        d = jnp.where(iota == im, jnp.inf, d)
    nn = jnp.concatenate(cols, axis=1)
    o_ref[0] = nn + b * NPTS


def _knn(pos, posT, rows=256):
    grid = (BZ, NPTS // rows)
    return pl.pallas_call(
        _knn_body,
        grid=grid,
        in_specs=[
            pl.BlockSpec((1, rows, 3), lambda b, i: (b, i, 0)),
            pl.BlockSpec((1, 3, NPTS), lambda b, i: (b, 0, 0)),
        ],
        out_specs=pl.BlockSpec((1, rows, KNB), lambda b, i: (b, i, 0)),
        out_shape=jax.ShapeDtypeStruct((BZ, NPTS, KNB), jnp.int32),
    )(pos, posT)


def _table_body(xp_ref, w1_ref, w1p_ref, b1_ref, t_ref, c_ref):
    xb = xp_ref[...]
    t_ref[...] = (
        jnp.dot(xb, w1_ref[...], preferred_element_type=F32) + b1_ref[...]
    )
    c_ref[...] = jnp.dot(
        xb[:, 125:128], w1p_ref[...], preferred_element_type=F32
    )


def _table(xp, W1, W1p, b1r, rows=512):
    npt = xp.shape[0]
    grid = (npt // rows,)
    return pl.pallas_call(
        _table_body,
        grid=grid,
        in_specs=[
            pl.BlockSpec((rows, 128), lambda i: (i, 0)),
            pl.BlockSpec((128, 256), lambda i: (0, 0)),
            pl.BlockSpec((3, 256), lambda i: (0, 0)),
            pl.BlockSpec((1, 256), lambda i: (0, 0)),
        ],
        out_specs=[
            pl.BlockSpec((rows, 256), lambda i: (i, 0)),
            pl.BlockSpec((rows, 256), lambda i: (i, 0)),
        ],
        out_shape=[
            jax.ShapeDtypeStruct((npt, 256), F32),
            jax.ShapeDtypeStruct((npt, 256), F32),
        ],
    )(xp, W1, W1p, b1r)


def _sc_gather(nn_flat, T):
    B = nn_flat.shape[0]
    D = T.shape[1]
    info = plsc.get_sparse_core_info()
    NC, NS = info.num_cores, info.num_subcores
    NW = NC * NS
    CH = 128
    b_per_w = B // NW
    iters = b_per_w // CH
    mesh = plsc.VectorSubcoreMesh(core_axis_name="c", subcore_axis_name="s")

    @functools.partial(
        pl.kernel,
        mesh=mesh,
        out_type=jax.ShapeDtypeStruct((B, D), F32),
        scratch_types=[
            pltpu.VMEM((CH,), jnp.int32),
            pltpu.VMEM((CH, D), F32),
            pltpu.SemaphoreType.DMA,
        ],
    )
    def gk(nn_hbm, t_hbm, out_hbm, idx_v, rows_v, sem):
        wid = lax.axis_index("s") * NC + lax.axis_index("c")
        base = wid * b_per_w

        def body(i, carry):
            off = base + i * CH
            pltpu.sync_copy(nn_hbm.at[pl.ds(off, CH)], idx_v)
            pltpu.async_copy(t_hbm.at[idx_v], rows_v, sem).wait()
            pltpu.sync_copy(rows_v, out_hbm.at[pl.ds(off, CH)])
            return carry

        lax.fori_loop(0, iters, body, 0)

    return gk(nn_flat, T)


def _stats1_body(g_ref, c_ref, p_ref):
    i = pl.program_id(0)
    rb = g_ref.shape[0]
    p = rb // KNB
    g3 = g_ref[...].reshape(p, KNB, 256)
    c3 = c_ref[...].reshape(p, 1, 256)
    h1 = (g3 - c3).reshape(rb, 256)

    @pl.when(i == 0)
    def _():
        p_ref[...] = jnp.zeros_like(p_ref)

    p_ref[0:1, :] += jnp.sum(h1, axis=0, keepdims=True)
    p_ref[1:2, :] += jnp.sum(h1 * h1, axis=0, keepdims=True)


def _stats1(G, C, rows=2048):
    B = G.shape[0]
    grid = (B // rows,)
    return pl.pallas_call(
        _stats1_body,
        grid=grid,
        in_specs=[
            pl.BlockSpec((rows, 256), lambda i: (i, 0)),
            pl.BlockSpec((rows // KNB, 256), lambda i: (i, 0)),
        ],
        out_specs=pl.BlockSpec((8, 256), lambda i: (0, 0)),
        out_shape=jax.ShapeDtypeStruct((8, 256), F32),
    )(G, C)


def _mlp_body(g_ref, c_ref, s1_ref, sh1_ref, w2_ref, b2_ref,
              mx_ref, mn_ref, p_ref):
    i = pl.program_id(0)
    rb = g_ref.shape[0]
    p = rb // KNB
    g3 = g_ref[...].reshape(p, KNB, 256)
    c3 = c_ref[...].reshape(p, 1, 256)
    h1 = (g3 - c3).reshape(rb, 256)
    a = jnp.maximum(h1 * s1_ref[...] + sh1_ref[...], 0.0)
    h2 = jnp.dot(a, w2_ref[...], preferred_element_type=F32) + b2_ref[...]

    @pl.when(i == 0)
    def _():
        p_ref[...] = jnp.zeros_like(p_ref)

    p_ref[0:1, :] += jnp.sum(h2, axis=0, keepdims=True)
    p_ref[1:2, :] += jnp.sum(h2 * h2, axis=0, keepdims=True)

    h23 = h2.reshape(p, KNB, 256)
    mx_ref[...] = jnp.max(h23, axis=1)
    mn_ref[...] = jnp.min(h23, axis=1)


def _mlp(G, C, scale1, shift1, W2, b2r, rows=2048):
    B = G.shape[0]
    npt = C.shape[0]
    grid = (B // rows,)
    return pl.pallas_call(
        _mlp_body,
        grid=grid,
        in_specs=[
            pl.BlockSpec((rows, 256), lambda i: (i, 0)),
            pl.BlockSpec((rows // KNB, 256), lambda i: (i, 0)),
            pl.BlockSpec((1, 256), lambda i: (0, 0)),
            pl.BlockSpec((1, 256), lambda i: (0, 0)),
            pl.BlockSpec((256, 256), lambda i: (0, 0)),
            pl.BlockSpec((1, 256), lambda i: (0, 0)),
        ],
        out_specs=[
            pl.BlockSpec((rows // KNB, 256), lambda i: (i, 0)),
            pl.BlockSpec((rows // KNB, 256), lambda i: (i, 0)),
            pl.BlockSpec((8, 256), lambda i: (0, 0)),
        ],
        out_shape=[
            jax.ShapeDtypeStruct((npt, 256), F32),
            jax.ShapeDtypeStruct((npt, 256), F32),
            jax.ShapeDtypeStruct((8, 256), F32),
        ],
    )(G, C, scale1, shift1, W2, b2r)


def _fin_body(mx_ref, mn_ref, s2_ref, sh2_ref, o_ref):
    s2 = s2_ref[...]
    pooled = jnp.where(s2 >= 0.0, mx_ref[...], mn_ref[...])
    o_ref[...] = pooled * s2 + sh2_ref[...]


def _finish(mx, mn, scale2, shift2, rows=512):
    npt = mx.shape[0]
    grid = (npt // rows,)
    return pl.pallas_call(
        _fin_body,
        grid=grid,
        in_specs=[
            pl.BlockSpec((rows, 256), lambda i: (i, 0)),
            pl.BlockSpec((rows, 256), lambda i: (i, 0)),
            pl.BlockSpec((1, 256), lambda i: (0, 0)),
            pl.BlockSpec((1, 256), lambda i: (0, 0)),
        ],
        out_specs=pl.BlockSpec((rows, 256), lambda i: (i, 0)),
        out_shape=jax.ShapeDtypeStruct((npt, 256), F32),
    )(mx, mn, scale2, shift2)


def _bn_affine(psums, cnt, g, beta):
    mean = psums[0] / cnt
    var = psums[1] / cnt - mean * mean
    scale = g / jnp.sqrt(var + EPSV)
    shift = beta - mean * scale
    return scale.reshape(1, 256), shift.reshape(1, 256)


def kernel(x, pos, n_sampling, W1, b1, g1, beta1, W2, b2, g2, beta2):
    del n_sampling
    bz, n, _ = x.shape
    npt = bz * n
    B = npt * KNB

    posT = pos.transpose(0, 2, 1)
    nn = _knn(pos, posT)
    nn_flat = nn.reshape(B)

    xp = jnp.concatenate([x, pos], axis=-1).reshape(npt, 128)
    W1p = W1[125:128]
    T, C = _table(xp, W1, W1p, b1.reshape(1, 256))

    G = _sc_gather(nn_flat, T)

    ps1 = _stats1(G, C)
    scale1, shift1 = _bn_affine(ps1, float(B), g1, beta1)

    mx, mn, ps2 = _mlp(G, C, scale1, shift1, W2, b2.reshape(1, 256))
    scale2, shift2 = _bn_affine(ps2, float(B), g2, beta2)

    out = _finish(mx, mn, scale2, shift2)
    return out.reshape(bz, n, 256)


# speedup vs baseline: 9.6483x; 1.0002x over previous
import functools

import jax
import jax.numpy as jnp
from jax import lax
from jax.experimental import pallas as pl
from jax.experimental.pallas import tpu as pltpu
from jax.experimental.pallas import tpu_sc as plsc

KNB = 16
NPTS = 2048
BZ = 4
EPSV = 1e-5
F32 = jnp.float32


def _knn_body(q_ref, aT_ref, o_ref):
    b = pl.program_id(0)
    pq = q_ref[0]
    pa = aT_ref[0]
    dx = pq[:, 0:1] - pa[0:1, :]
    dy = pq[:, 1:2] - pa[1:2, :]
    dz = pq[:, 2:3] - pa[2:3, :]
    d = (dx * dx + dy * dy) + dz * dz
    R = d.shape[0]
    S = 4
    rs = R // S
    iota = lax.broadcasted_iota(jnp.int32, (rs, NPTS), 1)
    ds = [d[s * rs:(s + 1) * rs] for s in range(S)]
    cols = [[] for _ in range(S)]
    for _ in range(KNB):
        for s in range(S):
            m = jnp.min(ds[s], axis=1, keepdims=True)
            im = jnp.min(jnp.where(ds[s] == m, iota, NPTS), axis=1,
                         keepdims=True)
            cols[s].append(im)
            ds[s] = jnp.where(iota == im, jnp.inf, ds[s])
    nn = jnp.concatenate(
        [jnp.concatenate(cols[s], axis=1) for s in range(S)], axis=0)
    o_ref[0] = nn + b * NPTS


def _knn(pos, posT, rows=256):
    grid = (BZ, NPTS // rows)
    return pl.pallas_call(
        _knn_body,
        grid=grid,
        in_specs=[
            pl.BlockSpec((1, rows, 3), lambda b, i: (b, i, 0)),
            pl.BlockSpec((1, 3, NPTS), lambda b, i: (b, 0, 0)),
        ],
        out_specs=pl.BlockSpec((1, rows, KNB), lambda b, i: (b, i, 0)),
        out_shape=jax.ShapeDtypeStruct((BZ, NPTS, KNB), jnp.int32),
    )(pos, posT)


def _table_body(xp_ref, w1_ref, w1p_ref, b1_ref, t_ref, c_ref):
    xb = xp_ref[...]
    t_ref[...] = (
        jnp.dot(xb, w1_ref[...], preferred_element_type=F32) + b1_ref[...]
    )
    c_ref[...] = jnp.dot(
        xb[:, 125:128], w1p_ref[...], preferred_element_type=F32
    )


def _table(xp, W1, W1p, b1r, rows=512):
    npt = xp.shape[0]
    grid = (npt // rows,)
    return pl.pallas_call(
        _table_body,
        grid=grid,
        in_specs=[
            pl.BlockSpec((rows, 128), lambda i: (i, 0)),
            pl.BlockSpec((128, 256), lambda i: (0, 0)),
            pl.BlockSpec((3, 256), lambda i: (0, 0)),
            pl.BlockSpec((1, 256), lambda i: (0, 0)),
        ],
        out_specs=[
            pl.BlockSpec((rows, 256), lambda i: (i, 0)),
            pl.BlockSpec((rows, 256), lambda i: (i, 0)),
        ],
        out_shape=[
            jax.ShapeDtypeStruct((npt, 256), F32),
            jax.ShapeDtypeStruct((npt, 256), F32),
        ],
    )(xp, W1, W1p, b1r)


def _sc_gather(nn_flat, T):
    B = nn_flat.shape[0]
    D = T.shape[1]
    info = plsc.get_sparse_core_info()
    NC, NS = info.num_cores, info.num_subcores
    NW = NC * NS
    CH = 128
    b_per_w = B // NW
    iters = b_per_w // CH
    mesh = plsc.VectorSubcoreMesh(core_axis_name="c", subcore_axis_name="s")

    @functools.partial(
        pl.kernel,
        mesh=mesh,
        out_type=jax.ShapeDtypeStruct((B, D), F32),
        scratch_types=[
            pltpu.VMEM((CH,), jnp.int32),
            pltpu.VMEM((CH, D), F32),
            pltpu.SemaphoreType.DMA,
        ],
    )
    def gk(nn_hbm, t_hbm, out_hbm, idx_v, rows_v, sem):
        wid = lax.axis_index("s") * NC + lax.axis_index("c")
        base = wid * b_per_w

        def body(i, carry):
            off = base + i * CH
            pltpu.sync_copy(nn_hbm.at[pl.ds(off, CH)], idx_v)
            pltpu.async_copy(t_hbm.at[idx_v], rows_v, sem).wait()
            pltpu.sync_copy(rows_v, out_hbm.at[pl.ds(off, CH)])
            return carry

        lax.fori_loop(0, iters, body, 0)

    return gk(nn_flat, T)


def _stats1_body(g_ref, c_ref, p_ref):
    i = pl.program_id(0)
    rb = g_ref.shape[0]
    p = rb // KNB
    g3 = g_ref[...].reshape(p, KNB, 256)
    c3 = c_ref[...].reshape(p, 1, 256)
    h1 = (g3 - c3).reshape(rb, 256)

    @pl.when(i == 0)
    def _():
        p_ref[...] = jnp.zeros_like(p_ref)

    p_ref[0:1, :] += jnp.sum(h1, axis=0, keepdims=True)
    p_ref[1:2, :] += jnp.sum(h1 * h1, axis=0, keepdims=True)


def _stats1(G, C, rows=2048):
    B = G.shape[0]
    grid = (B // rows,)
    return pl.pallas_call(
        _stats1_body,
        grid=grid,
        in_specs=[
            pl.BlockSpec((rows, 256), lambda i: (i, 0)),
            pl.BlockSpec((rows // KNB, 256), lambda i: (i, 0)),
        ],
        out_specs=pl.BlockSpec((8, 256), lambda i: (0, 0)),
        out_shape=jax.ShapeDtypeStruct((8, 256), F32),
    )(G, C)


def _mlp_body(g_ref, c_ref, s1_ref, sh1_ref, w2_ref, b2_ref,
              mx_ref, mn_ref, p_ref):
    i = pl.program_id(0)
    rb = g_ref.shape[0]
    p = rb // KNB
    g3 = g_ref[...].reshape(p, KNB, 256)
    c3 = c_ref[...].reshape(p, 1, 256)
    h1 = (g3 - c3).reshape(rb, 256)
    a = jnp.maximum(h1 * s1_ref[...] + sh1_ref[...], 0.0)
    h2 = jnp.dot(a, w2_ref[...], preferred_element_type=F32) + b2_ref[...]

    @pl.when(i == 0)
    def _():
        p_ref[...] = jnp.zeros_like(p_ref)

    p_ref[0:1, :] += jnp.sum(h2, axis=0, keepdims=True)
    p_ref[1:2, :] += jnp.sum(h2 * h2, axis=0, keepdims=True)

    h23 = h2.reshape(p, KNB, 256)
    mx_ref[...] = jnp.max(h23, axis=1)
    mn_ref[...] = jnp.min(h23, axis=1)


def _mlp(G, C, scale1, shift1, W2, b2r, rows=2048):
    B = G.shape[0]
    npt = C.shape[0]
    grid = (B // rows,)
    return pl.pallas_call(
        _mlp_body,
        grid=grid,
        in_specs=[
            pl.BlockSpec((rows, 256), lambda i: (i, 0)),
            pl.BlockSpec((rows // KNB, 256), lambda i: (i, 0)),
            pl.BlockSpec((1, 256), lambda i: (0, 0)),
            pl.BlockSpec((1, 256), lambda i: (0, 0)),
            pl.BlockSpec((256, 256), lambda i: (0, 0)),
            pl.BlockSpec((1, 256), lambda i: (0, 0)),
        ],
        out_specs=[
            pl.BlockSpec((rows // KNB, 256), lambda i: (i, 0)),
            pl.BlockSpec((rows // KNB, 256), lambda i: (i, 0)),
            pl.BlockSpec((8, 256), lambda i: (0, 0)),
        ],
        out_shape=[
            jax.ShapeDtypeStruct((npt, 256), F32),
            jax.ShapeDtypeStruct((npt, 256), F32),
            jax.ShapeDtypeStruct((8, 256), F32),
        ],
    )(G, C, scale1, shift1, W2, b2r)


def _fin_body(mx_ref, mn_ref, s2_ref, sh2_ref, o_ref):
    s2 = s2_ref[...]
    pooled = jnp.where(s2 >= 0.0, mx_ref[...], mn_ref[...])
    o_ref[...] = pooled * s2 + sh2_ref[...]


def _finish(mx, mn, scale2, shift2, rows=512):
    npt = mx.shape[0]
    grid = (npt // rows,)
    return pl.pallas_call(
        _fin_body,
        grid=grid,
        in_specs=[
            pl.BlockSpec((rows, 256), lambda i: (i, 0)),
            pl.BlockSpec((rows, 256), lambda i: (i, 0)),
            pl.BlockSpec((1, 256), lambda i: (0, 0)),
            pl.BlockSpec((1, 256), lambda i: (0, 0)),
        ],
        out_specs=pl.BlockSpec((rows, 256), lambda i: (i, 0)),
        out_shape=jax.ShapeDtypeStruct((npt, 256), F32),
    )(mx, mn, scale2, shift2)


def _bn_affine(psums, cnt, g, beta):
    mean = psums[0] / cnt
    var = psums[1] / cnt - mean * mean
    scale = g / jnp.sqrt(var + EPSV)
    shift = beta - mean * scale
    return scale.reshape(1, 256), shift.reshape(1, 256)


def kernel(x, pos, n_sampling, W1, b1, g1, beta1, W2, b2, g2, beta2):
    del n_sampling
    bz, n, _ = x.shape
    npt = bz * n
    B = npt * KNB

    posT = pos.transpose(0, 2, 1)
    nn = _knn(pos, posT)
    nn_flat = nn.reshape(B)

    xp = jnp.concatenate([x, pos], axis=-1).reshape(npt, 128)
    W1p = W1[125:128]
    T, C = _table(xp, W1, W1p, b1.reshape(1, 256))

    G = _sc_gather(nn_flat, T)

    ps1 = _stats1(G, C)
    scale1, shift1 = _bn_affine(ps1, float(B), g1, beta1)

    mx, mn, ps2 = _mlp(G, C, scale1, shift1, W2, b2.reshape(1, 256))
    scale2, shift2 = _bn_affine(ps2, float(B), g2, beta2)

    out = _finish(mx, mn, scale2, shift2)
    return out.reshape(bz, n, 256)


# speedup vs baseline: 10.8746x; 1.1271x over previous
import functools

import jax
import jax.numpy as jnp
from jax import lax
from jax.experimental import pallas as pl
from jax.experimental.pallas import tpu as pltpu
from jax.experimental.pallas import tpu_sc as plsc

KNB = 16
NPTS = 2048
BZ = 4
EPSV = 1e-5
F32 = jnp.float32


def _knn_body(q_ref, aT_ref, o_ref):
    b = pl.program_id(0)
    pq = q_ref[0]
    pa = aT_ref[0]
    dx = pq[:, 0:1] - pa[0:1, :]
    dy = pq[:, 1:2] - pa[1:2, :]
    dz = pq[:, 2:3] - pa[2:3, :]
    d = (dx * dx + dy * dy) + dz * dz
    R = d.shape[0]
    S = 4
    rs = R // S
    iotaf = lax.broadcasted_iota(jnp.int32, (rs, NPTS), 1).astype(F32)
    big = F32(2.0 * NPTS)
    ds = [d[s * rs:(s + 1) * rs] for s in range(S)]
    cols = [[] for _ in range(S)]
    for _ in range(KNB):
        for s in range(S):
            m = jnp.min(ds[s], axis=1, keepdims=True)
            im = jnp.min(jnp.where(ds[s] == m, iotaf, big), axis=1,
                         keepdims=True)
            cols[s].append(im)
            ds[s] = jnp.where(iotaf == im, jnp.inf, ds[s])
    nn = jnp.concatenate(
        [jnp.concatenate(cols[s], axis=1) for s in range(S)],
        axis=0).astype(jnp.int32)
    o_ref[0] = nn + b * NPTS


def _knn(pos, posT, rows=256):
    grid = (BZ, NPTS // rows)
    return pl.pallas_call(
        _knn_body,
        grid=grid,
        in_specs=[
            pl.BlockSpec((1, rows, 3), lambda b, i: (b, i, 0)),
            pl.BlockSpec((1, 3, NPTS), lambda b, i: (b, 0, 0)),
        ],
        out_specs=pl.BlockSpec((1, rows, KNB), lambda b, i: (b, i, 0)),
        out_shape=jax.ShapeDtypeStruct((BZ, NPTS, KNB), jnp.int32),
    )(pos, posT)


def _table_body(xp_ref, w1_ref, w1p_ref, b1_ref, t_ref, c_ref):
    xb = xp_ref[...]
    t_ref[...] = (
        jnp.dot(xb, w1_ref[...], preferred_element_type=F32) + b1_ref[...]
    )
    c_ref[...] = jnp.dot(
        xb[:, 125:128], w1p_ref[...], preferred_element_type=F32
    )


def _table(xp, W1, W1p, b1r, rows=512):
    npt = xp.shape[0]
    grid = (npt // rows,)
    return pl.pallas_call(
        _table_body,
        grid=grid,
        in_specs=[
            pl.BlockSpec((rows, 128), lambda i: (i, 0)),
            pl.BlockSpec((128, 256), lambda i: (0, 0)),
            pl.BlockSpec((3, 256), lambda i: (0, 0)),
            pl.BlockSpec((1, 256), lambda i: (0, 0)),
        ],
        out_specs=[
            pl.BlockSpec((rows, 256), lambda i: (i, 0)),
            pl.BlockSpec((rows, 256), lambda i: (i, 0)),
        ],
        out_shape=[
            jax.ShapeDtypeStruct((npt, 256), F32),
            jax.ShapeDtypeStruct((npt, 256), F32),
        ],
    )(xp, W1, W1p, b1r)


def _sc_gather(nn_flat, T):
    B = nn_flat.shape[0]
    D = T.shape[1]
    info = plsc.get_sparse_core_info()
    NC, NS = info.num_cores, info.num_subcores
    NW = NC * NS
    CH = 128
    b_per_w = B // NW
    iters = b_per_w // CH
    mesh = plsc.VectorSubcoreMesh(core_axis_name="c", subcore_axis_name="s")

    @functools.partial(
        pl.kernel,
        mesh=mesh,
        out_type=jax.ShapeDtypeStruct((B, D), F32),
        scratch_types=[
            pltpu.VMEM((CH,), jnp.int32),
            pltpu.VMEM((CH, D), F32),
            pltpu.SemaphoreType.DMA,
        ],
    )
    def gk(nn_hbm, t_hbm, out_hbm, idx_v, rows_v, sem):
        wid = lax.axis_index("s") * NC + lax.axis_index("c")
        base = wid * b_per_w

        def body(i, carry):
            off = base + i * CH
            pltpu.sync_copy(nn_hbm.at[pl.ds(off, CH)], idx_v)
            pltpu.async_copy(t_hbm.at[idx_v], rows_v, sem).wait()
            pltpu.sync_copy(rows_v, out_hbm.at[pl.ds(off, CH)])
            return carry

        lax.fori_loop(0, iters, body, 0)

    return gk(nn_flat, T)


def _stats1_body(g_ref, c_ref, p_ref):
    i = pl.program_id(0)
    rb = g_ref.shape[0]
    p = rb // KNB
    g3 = g_ref[...].reshape(p, KNB, 256)
    c3 = c_ref[...].reshape(p, 1, 256)
    h1 = (g3 - c3).reshape(rb, 256)

    @pl.when(i == 0)
    def _():
        p_ref[...] = jnp.zeros_like(p_ref)

    p_ref[0:1, :] += jnp.sum(h1, axis=0, keepdims=True)
    p_ref[1:2, :] += jnp.sum(h1 * h1, axis=0, keepdims=True)


def _stats1(G, C, rows=2048):
    B = G.shape[0]
    grid = (B // rows,)
    return pl.pallas_call(
        _stats1_body,
        grid=grid,
        in_specs=[
            pl.BlockSpec((rows, 256), lambda i: (i, 0)),
            pl.BlockSpec((rows // KNB, 256), lambda i: (i, 0)),
        ],
        out_specs=pl.BlockSpec((8, 256), lambda i: (0, 0)),
        out_shape=jax.ShapeDtypeStruct((8, 256), F32),
    )(G, C)


def _mlp_body(g_ref, c_ref, s1_ref, sh1_ref, w2_ref, b2_ref,
              mx_ref, mn_ref, p_ref):
    i = pl.program_id(0)
    rb = g_ref.shape[0]
    p = rb // KNB
    g3 = g_ref[...].reshape(p, KNB, 256)
    c3 = c_ref[...].reshape(p, 1, 256)
    h1 = (g3 - c3).reshape(rb, 256)
    a = jnp.maximum(h1 * s1_ref[...] + sh1_ref[...], 0.0)
    h2 = jnp.dot(a, w2_ref[...], preferred_element_type=F32) + b2_ref[...]

    @pl.when(i == 0)
    def _():
        p_ref[...] = jnp.zeros_like(p_ref)

    p_ref[0:1, :] += jnp.sum(h2, axis=0, keepdims=True)
    p_ref[1:2, :] += jnp.sum(h2 * h2, axis=0, keepdims=True)

    h23 = h2.reshape(p, KNB, 256)
    mx_ref[...] = jnp.max(h23, axis=1)
    mn_ref[...] = jnp.min(h23, axis=1)


def _mlp(G, C, scale1, shift1, W2, b2r, rows=2048):
    B = G.shape[0]
    npt = C.shape[0]
    grid = (B // rows,)
    return pl.pallas_call(
        _mlp_body,
        grid=grid,
        in_specs=[
            pl.BlockSpec((rows, 256), lambda i: (i, 0)),
            pl.BlockSpec((rows // KNB, 256), lambda i: (i, 0)),
            pl.BlockSpec((1, 256), lambda i: (0, 0)),
            pl.BlockSpec((1, 256), lambda i: (0, 0)),
            pl.BlockSpec((256, 256), lambda i: (0, 0)),
            pl.BlockSpec((1, 256), lambda i: (0, 0)),
        ],
        out_specs=[
            pl.BlockSpec((rows // KNB, 256), lambda i: (i, 0)),
            pl.BlockSpec((rows // KNB, 256), lambda i: (i, 0)),
            pl.BlockSpec((8, 256), lambda i: (0, 0)),
        ],
        out_shape=[
            jax.ShapeDtypeStruct((npt, 256), F32),
            jax.ShapeDtypeStruct((npt, 256), F32),
            jax.ShapeDtypeStruct((8, 256), F32),
        ],
    )(G, C, scale1, shift1, W2, b2r)


def _fin_body(mx_ref, mn_ref, s2_ref, sh2_ref, o_ref):
    s2 = s2_ref[...]
    pooled = jnp.where(s2 >= 0.0, mx_ref[...], mn_ref[...])
    o_ref[...] = pooled * s2 + sh2_ref[...]


def _finish(mx, mn, scale2, shift2, rows=512):
    npt = mx.shape[0]
    grid = (npt // rows,)
    return pl.pallas_call(
        _fin_body,
        grid=grid,
        in_specs=[
            pl.BlockSpec((rows, 256), lambda i: (i, 0)),
            pl.BlockSpec((rows, 256), lambda i: (i, 0)),
            pl.BlockSpec((1, 256), lambda i: (0, 0)),
            pl.BlockSpec((1, 256), lambda i: (0, 0)),
        ],
        out_specs=pl.BlockSpec((rows, 256), lambda i: (i, 0)),
        out_shape=jax.ShapeDtypeStruct((npt, 256), F32),
    )(mx, mn, scale2, shift2)


def _bn_affine(psums, cnt, g, beta):
    mean = psums[0] / cnt
    var = psums[1] / cnt - mean * mean
    scale = g / jnp.sqrt(var + EPSV)
    shift = beta - mean * scale
    return scale.reshape(1, 256), shift.reshape(1, 256)


def kernel(x, pos, n_sampling, W1, b1, g1, beta1, W2, b2, g2, beta2):
    del n_sampling
    bz, n, _ = x.shape
    npt = bz * n
    B = npt * KNB

    posT = pos.transpose(0, 2, 1)
    nn = _knn(pos, posT)
    nn_flat = nn.reshape(B)

    xp = jnp.concatenate([x, pos], axis=-1).reshape(npt, 128)
    W1p = W1[125:128]
    T, C = _table(xp, W1, W1p, b1.reshape(1, 256))

    G = _sc_gather(nn_flat, T)

    ps1 = _stats1(G, C)
    scale1, shift1 = _bn_affine(ps1, float(B), g1, beta1)

    mx, mn, ps2 = _mlp(G, C, scale1, shift1, W2, b2.reshape(1, 256))
    scale2, shift2 = _bn_affine(ps2, float(B), g2, beta2)

    out = _finish(mx, mn, scale2, shift2)
    return out.reshape(bz, n, 256)


# speedup vs baseline: 12.3982x; 1.1401x over previous
import functools

import jax
import jax.numpy as jnp
from jax import lax
from jax.experimental import pallas as pl
from jax.experimental.pallas import tpu as pltpu
from jax.experimental.pallas import tpu_sc as plsc

KNB = 16
NPTS = 2048
BZ = 4
EPSV = 1e-5
F32 = jnp.float32


def _knn_body(base, q_ref, aT_ref, o_ref):
    pq = q_ref[...]
    pa = aT_ref[...]
    dx = pq[:, 0:1] - pa[0:1, :]
    dy = pq[:, 1:2] - pa[1:2, :]
    dz = pq[:, 2:3] - pa[2:3, :]
    d = (dx * dx + dy * dy) + dz * dz
    R = d.shape[0]
    S = 4
    rs = R // S
    iotaf = lax.broadcasted_iota(jnp.int32, (rs, NPTS), 1).astype(F32)
    big = F32(2.0 * NPTS)
    ds = [d[s * rs:(s + 1) * rs] for s in range(S)]
    cols = [[] for _ in range(S)]
    for _ in range(KNB):
        for s in range(S):
            m = jnp.min(ds[s], axis=1, keepdims=True)
            im = jnp.min(jnp.where(ds[s] == m, iotaf, big), axis=1,
                         keepdims=True)
            cols[s].append(im)
            ds[s] = jnp.where(iotaf == im, jnp.inf, ds[s])
    nn = jnp.concatenate(
        [jnp.concatenate(cols[s], axis=1) for s in range(S)],
        axis=0).astype(jnp.int32)
    o_ref[...] = nn + base


def _knn_one(pos_b, posT_b, base, rows=256):
    grid = (NPTS // rows,)
    return pl.pallas_call(
        functools.partial(_knn_body, base),
        grid=grid,
        in_specs=[
            pl.BlockSpec((rows, 3), lambda i: (i, 0)),
            pl.BlockSpec((3, NPTS), lambda i: (0, 0)),
        ],
        out_specs=pl.BlockSpec((rows, KNB), lambda i: (i, 0)),
        out_shape=jax.ShapeDtypeStruct((NPTS, KNB), jnp.int32),
    )(pos_b, posT_b)


def _table_body(xp_ref, w1_ref, w1p_ref, b1_ref, t_ref, c_ref):
    xb = xp_ref[...]
    t_ref[...] = (
        jnp.dot(xb, w1_ref[...], preferred_element_type=F32) + b1_ref[...]
    )
    c_ref[...] = jnp.dot(
        xb[:, 125:128], w1p_ref[...], preferred_element_type=F32
    )


def _table(xp, W1, W1p, b1r, rows=512):
    npt = xp.shape[0]
    grid = (npt // rows,)
    return pl.pallas_call(
        _table_body,
        grid=grid,
        in_specs=[
            pl.BlockSpec((rows, 128), lambda i: (i, 0)),
            pl.BlockSpec((128, 256), lambda i: (0, 0)),
            pl.BlockSpec((3, 256), lambda i: (0, 0)),
            pl.BlockSpec((1, 256), lambda i: (0, 0)),
        ],
        out_specs=[
            pl.BlockSpec((rows, 256), lambda i: (i, 0)),
            pl.BlockSpec((rows, 256), lambda i: (i, 0)),
        ],
        out_shape=[
            jax.ShapeDtypeStruct((npt, 256), F32),
            jax.ShapeDtypeStruct((npt, 256), F32),
        ],
    )(xp, W1, W1p, b1r)


def _sc_gather(nn_flat, T):
    B = nn_flat.shape[0]
    D = T.shape[1]
    info = plsc.get_sparse_core_info()
    NC, NS = info.num_cores, info.num_subcores
    NW = NC * NS
    CH = 128
    b_per_w = B // NW
    iters = b_per_w // CH
    mesh = plsc.VectorSubcoreMesh(core_axis_name="c", subcore_axis_name="s")

    @functools.partial(
        pl.kernel,
        mesh=mesh,
        out_type=jax.ShapeDtypeStruct((B, D), F32),
        scratch_types=[
            pltpu.VMEM((CH,), jnp.int32),
            pltpu.VMEM((CH, D), F32),
            pltpu.SemaphoreType.DMA,
        ],
    )
    def gk(nn_hbm, t_hbm, out_hbm, idx_v, rows_v, sem):
        wid = lax.axis_index("s") * NC + lax.axis_index("c")
        base = wid * b_per_w

        def body(i, carry):
            off = base + i * CH
            pltpu.sync_copy(nn_hbm.at[pl.ds(off, CH)], idx_v)
            pltpu.async_copy(t_hbm.at[idx_v], rows_v, sem).wait()
            pltpu.sync_copy(rows_v, out_hbm.at[pl.ds(off, CH)])
            return carry

        lax.fori_loop(0, iters, body, 0)

    return gk(nn_flat, T)


def _stats1_body(g_ref, c_ref, p_ref):
    i = pl.program_id(0)
    rb = g_ref.shape[0]
    p = rb // KNB
    g3 = g_ref[...].reshape(p, KNB, 256)
    c3 = c_ref[...].reshape(p, 1, 256)
    h1 = (g3 - c3).reshape(rb, 256)

    @pl.when(i == 0)
    def _():
        p_ref[...] = jnp.zeros_like(p_ref)

    p_ref[0:1, :] += jnp.sum(h1, axis=0, keepdims=True)
    p_ref[1:2, :] += jnp.sum(h1 * h1, axis=0, keepdims=True)


def _stats1(G, C, rows=2048):
    B = G.shape[0]
    grid = (B // rows,)
    return pl.pallas_call(
        _stats1_body,
        grid=grid,
        in_specs=[
            pl.BlockSpec((rows, 256), lambda i: (i, 0)),
            pl.BlockSpec((rows // KNB, 256), lambda i: (i, 0)),
        ],
        out_specs=pl.BlockSpec((8, 256), lambda i: (0, 0)),
        out_shape=jax.ShapeDtypeStruct((8, 256), F32),
    )(G, C)


def _mlp_body(g_ref, c_ref, s1_ref, sh1_ref, w2_ref, b2_ref,
              mx_ref, mn_ref, p_ref):
    i = pl.program_id(0)
    rb = g_ref.shape[0]
    p = rb // KNB
    g3 = g_ref[...].reshape(p, KNB, 256)
    c3 = c_ref[...].reshape(p, 1, 256)
    h1 = (g3 - c3).reshape(rb, 256)
    a = jnp.maximum(h1 * s1_ref[...] + sh1_ref[...], 0.0)
    h2 = jnp.dot(a, w2_ref[...], preferred_element_type=F32) + b2_ref[...]

    @pl.when(i == 0)
    def _():
        p_ref[...] = jnp.zeros_like(p_ref)

    p_ref[0:1, :] += jnp.sum(h2, axis=0, keepdims=True)
    p_ref[1:2, :] += jnp.sum(h2 * h2, axis=0, keepdims=True)

    h23 = h2.reshape(p, KNB, 256)
    mx_ref[...] = jnp.max(h23, axis=1)
    mn_ref[...] = jnp.min(h23, axis=1)


def _mlp(G, C, scale1, shift1, W2, b2r, rows=2048):
    B = G.shape[0]
    npt = C.shape[0]
    grid = (B // rows,)
    return pl.pallas_call(
        _mlp_body,
        grid=grid,
        in_specs=[
            pl.BlockSpec((rows, 256), lambda i: (i, 0)),
            pl.BlockSpec((rows // KNB, 256), lambda i: (i, 0)),
            pl.BlockSpec((1, 256), lambda i: (0, 0)),
            pl.BlockSpec((1, 256), lambda i: (0, 0)),
            pl.BlockSpec((256, 256), lambda i: (0, 0)),
            pl.BlockSpec((1, 256), lambda i: (0, 0)),
        ],
        out_specs=[
            pl.BlockSpec((rows // KNB, 256), lambda i: (i, 0)),
            pl.BlockSpec((rows // KNB, 256), lambda i: (i, 0)),
            pl.BlockSpec((8, 256), lambda i: (0, 0)),
        ],
        out_shape=[
            jax.ShapeDtypeStruct((npt, 256), F32),
            jax.ShapeDtypeStruct((npt, 256), F32),
            jax.ShapeDtypeStruct((8, 256), F32),
        ],
    )(G, C, scale1, shift1, W2, b2r)


def _fin_body(mx_ref, mn_ref, s2_ref, sh2_ref, o_ref):
    s2 = s2_ref[...]
    pooled = jnp.where(s2 >= 0.0, mx_ref[...], mn_ref[...])
    o_ref[...] = pooled * s2 + sh2_ref[...]


def _finish(mx, mn, scale2, shift2, rows=512):
    npt = mx.shape[0]
    grid = (npt // rows,)
    return pl.pallas_call(
        _fin_body,
        grid=grid,
        in_specs=[
            pl.BlockSpec((rows, 256), lambda i: (i, 0)),
            pl.BlockSpec((rows, 256), lambda i: (i, 0)),
            pl.BlockSpec((1, 256), lambda i: (0, 0)),
            pl.BlockSpec((1, 256), lambda i: (0, 0)),
        ],
        out_specs=pl.BlockSpec((rows, 256), lambda i: (i, 0)),
        out_shape=jax.ShapeDtypeStruct((npt, 256), F32),
    )(mx, mn, scale2, shift2)


def _bn_affine(psums, cnt, g, beta):
    mean = psums[0] / cnt
    var = psums[1] / cnt - mean * mean
    scale = g / jnp.sqrt(var + EPSV)
    shift = beta - mean * scale
    return scale.reshape(1, 256), shift.reshape(1, 256)


def kernel(x, pos, n_sampling, W1, b1, g1, beta1, W2, b2, g2, beta2):
    del n_sampling
    bz, n, _ = x.shape
    npt = bz * n
    B = npt * KNB

    posT = pos.transpose(0, 2, 1)

    xp = jnp.concatenate([x, pos], axis=-1).reshape(npt, 128)
    W1p = W1[125:128]
    T, C = _table(xp, W1, W1p, b1.reshape(1, 256))

    nns = [_knn_one(pos[b], posT[b], b * NPTS) for b in range(bz)]
    Gs = [_sc_gather(nns[b].reshape(n * KNB), T) for b in range(bz)]
    Cs = [C[b * n:(b + 1) * n] for b in range(bz)]

    ps1 = sum(_stats1(Gs[b], Cs[b]) for b in range(bz))
    scale1, shift1 = _bn_affine(ps1, float(B), g1, beta1)

    b2r = b2.reshape(1, 256)
    mlps = [_mlp(Gs[b], Cs[b], scale1, shift1, W2, b2r) for b in range(bz)]
    ps2 = sum(m[2] for m in mlps)
    scale2, shift2 = _bn_affine(ps2, float(B), g2, beta2)

    outs = [_finish(m[0], m[1], scale2, shift2) for m in mlps]
    return jnp.stack(outs, axis=0)
